# baseline (speedup 1.0000x reference)
import functools

import numpy as np
import jax
import jax.numpy as jnp
from jax import lax
from jax.experimental import pallas as pl
from jax.experimental.pallas import tpu as pltpu
from jax.experimental.pallas import tpu_sc as plsc

EMB_DIM = 64
SPE1_IDX = 2
SPE2_IDX = 3
SEQ = 200
BATCH = 1024
TOK = BATCH * SEQ
NPROWS = 80
TOTAL_IDX = TOK + NPROWS

NC, NS = 2, 16
NW = NC * NS
GSZ = 128
CHUNK = 2 * GSZ
K = 26
BPW = CHUNK * K
BPAD = BPW * NW
BPAD2 = BPAD // 2
PROW2 = TOK // 2

RBLK = 1600
GRID = PROW2 // RBLK


def _positional_encoding(L, d):
    position = np.arange(L, dtype=np.float32)[:, None]
    div_term = np.exp(np.arange(0, d, 2, dtype=np.float32) * (-np.log(10000.0) / d))
    pe = np.zeros((L, d), dtype=np.float32)
    pe[:, 0::2] = np.sin(position * div_term)
    pe[:, 1::2] = np.cos(position * div_term)
    return pe


def _pe_packed():
    pe = _positional_encoding(SEQ, EMB_DIM)
    pos = np.arange(RBLK * 2) % SEQ
    flat = pe[pos]
    return flat.reshape(RBLK, 2 * EMB_DIM)


_PE2 = _pe_packed()


def _sc_gather(emb_table, idx):
    mesh = plsc.VectorSubcoreMesh(core_axis_name="c", subcore_axis_name="s")

    @functools.partial(
        pl.kernel,
        mesh=mesh,
        out_type=jax.ShapeDtypeStruct((BPAD2, 2 * EMB_DIM), jnp.float32),
        scratch_types=[
            pltpu.VMEM((K, 2, GSZ), jnp.int32),
            pltpu.VMEM((2, 2, GSZ, EMB_DIM), jnp.float32),
            pltpu.SemaphoreType.DMA,
        ],
        compiler_params=pltpu.CompilerParams(use_tc_tiling_on_sc=False),
    )
    def gather_kernel(table_hbm, idx_hbm, out_hbm, idx_v, bufs, gsem):
        wid = lax.axis_index("s") * NC + lax.axis_index("c")
        base2 = wid * (BPW // 2)
        pltpu.sync_copy(idx_hbm.at[wid], idx_v)

        def fire(k, b):
            pltpu.async_copy(table_hbm.at[idx_v.at[k, 0]], bufs.at[b, 0], gsem)
            pltpu.async_copy(table_hbm.at[idx_v.at[k, 1]], bufs.at[b, 1], gsem)

        def drain_write(k, b):
            pltpu.make_async_copy(table_hbm.at[idx_v.at[k, 0]], bufs.at[b, 0], gsem).wait()
            pltpu.make_async_copy(table_hbm.at[idx_v.at[k, 1]], bufs.at[b, 1], gsem).wait()
            rb = base2 + k * GSZ
            pltpu.sync_copy(bufs.at[b, 0], out_hbm.at[pl.ds(rb, GSZ), pl.ds(0, EMB_DIM)])
            pltpu.sync_copy(bufs.at[b, 1], out_hbm.at[pl.ds(rb, GSZ), pl.ds(EMB_DIM, EMB_DIM)])

        fire(0, 0)

        def body(k2, carry):
            for b in (0, 1):
                k = 2 * k2 + b

                @pl.when(k + 1 < K)
                def _():
                    fire(k + 1, 1 - b)

                drain_write(k, b)
            return carry

        lax.fori_loop(0, K // 2, body, 0)

    return gather_kernel(emb_table, idx)


def _tc_body(emb_ref, seg_ref, prow_ref, pe_ref, w_ref, b_ref, out_ref):
    ps = prow_ref[...]
    p0 = jnp.sum(ps[0:20, :], axis=0, keepdims=True)
    p1 = jnp.sum(ps[20:40, :], axis=0, keepdims=True)
    zeros = jnp.zeros((1, EMB_DIM), jnp.float32)
    p0s = p0[:, 0:EMB_DIM] + p0[:, EMB_DIM:]
    p1s = p1[:, 0:EMB_DIM] + p1[:, EMB_DIM:]
    p0l = jnp.concatenate([p0s, zeros], axis=1)
    p0r = jnp.concatenate([zeros, p0s], axis=1)
    p1l = jnp.concatenate([p1s, zeros], axis=1)
    p1r = jnp.concatenate([zeros, p1s], axis=1)

    seg = seg_ref[...]
    m0e = (seg[:, 0:1] == SPE1_IDX).astype(jnp.float32)
    m0o = (seg[:, 1:2] == SPE1_IDX).astype(jnp.float32)
    m1e = (seg[:, 0:1] == SPE2_IDX).astype(jnp.float32)
    m1o = (seg[:, 1:2] == SPE2_IDX).astype(jnp.float32)

    emb = emb_ref[...] * np.float32(8.0)
    emb = emb + m0e * p0l + m0o * p0r + m1e * p1l + m1o * p1r + pe_ref[...]
    out_ref[...] = (
        jnp.dot(emb, w_ref[...], preferred_element_type=jnp.float32) + b_ref[...]
    )


def _tc_project(packed, seg2, w2, b2, pe2):
    return pl.pallas_call(
        _tc_body,
        grid=(GRID,),
        in_specs=[
            pl.BlockSpec((RBLK, 2 * EMB_DIM), lambda i: (i, 0)),
            pl.BlockSpec((RBLK, 2), lambda i: (i, 0)),
            pl.BlockSpec((NPROWS // 2, 2 * EMB_DIM), lambda i: (PROW2 // (NPROWS // 2), 0)),
            pl.BlockSpec((RBLK, 2 * EMB_DIM), lambda i: (0, 0)),
            pl.BlockSpec((2 * EMB_DIM, 1024), lambda i: (0, 0)),
            pl.BlockSpec((1, 1024), lambda i: (0, 0)),
        ],
        out_specs=pl.BlockSpec((RBLK, 1024), lambda i: (i, 0)),
        out_shape=jax.ShapeDtypeStruct((PROW2, 1024), jnp.float32),
    )(packed, seg2, packed, pe2, w2, b2)


def kernel(context, segs, personas_no_tag, tags, emb_table, proj_w, proj_b):
    idx_full = jnp.concatenate([
        context.reshape(-1),
        personas_no_tag[0], tags[0],
        personas_no_tag[1], tags[1],
    ]).astype(jnp.int32)
    idx_pad = jnp.concatenate(
        [idx_full, jnp.zeros((BPAD - TOTAL_IDX,), jnp.int32)]
    ).reshape(NW, K, GSZ, 2)
    idx_eo = jnp.swapaxes(idx_pad, 2, 3)

    packed = _sc_gather(emb_table, idx_eo)

    seg2 = segs.reshape(TOK // 2, 2).astype(jnp.int32)
    w2 = jnp.zeros((2 * EMB_DIM, 1024), jnp.float32)
    w2 = w2.at[0:EMB_DIM, 0:512].set(proj_w).at[EMB_DIM:, 512:].set(proj_w)
    b2 = jnp.tile(proj_b.reshape(1, 512), (1, 2))

    out = _tc_project(packed, seg2, w2, b2, jnp.asarray(_PE2))
    return out.reshape(BATCH, SEQ, 512)


# speedup vs baseline: 1.4890x; 1.4890x over previous
import functools

import numpy as np
import jax
import jax.numpy as jnp
from jax import lax
from jax.experimental import pallas as pl
from jax.experimental.pallas import tpu as pltpu
from jax.experimental.pallas import tpu_sc as plsc

EMB_DIM = 64
SPE1_IDX = 2
SPE2_IDX = 3
SEQ = 200
BATCH = 1024
TOK = BATCH * SEQ
NPROWS = 80

NC, NS = 2, 16
NW = NC * NS
GSZ = 128
K = 50
BPW = GSZ * K
NB = 4

RBLK = 3200
GRID = TOK // RBLK


def _positional_encoding(L, d):
    position = np.arange(L, dtype=np.float32)[:, None]
    div_term = np.exp(np.arange(0, d, 2, dtype=np.float32) * (-np.log(10000.0) / d))
    pe = np.zeros((L, d), dtype=np.float32)
    pe[:, 0::2] = np.sin(position * div_term)
    pe[:, 1::2] = np.cos(position * div_term)
    return pe


_PE_REP = np.tile(_positional_encoding(SEQ, EMB_DIM), (RBLK // SEQ, 1))


def _sc_gather(emb_table, idx, idx_p):
    mesh = plsc.VectorSubcoreMesh(core_axis_name="c", subcore_axis_name="s")

    @functools.partial(
        pl.kernel,
        mesh=mesh,
        out_type=(
            jax.ShapeDtypeStruct((TOK, EMB_DIM), jnp.float32),
            jax.ShapeDtypeStruct((NPROWS, EMB_DIM), jnp.float32),
        ),
        scratch_types=[
            pltpu.VMEM((BPW,), jnp.int32),
            pltpu.VMEM((NB, GSZ, EMB_DIM), jnp.float32),
            pltpu.VMEM((NPROWS,), jnp.int32),
            pltpu.VMEM((NPROWS, EMB_DIM), jnp.float32),
            pltpu.SemaphoreType.DMA,
        ],
        compiler_params=pltpu.CompilerParams(use_tc_tiling_on_sc=False),
    )
    def gather_kernel(table_hbm, idx_hbm, idxp_hbm, out_hbm, outp_hbm,
                      idx_v, bufs, idxp_v, pbuf, gsem):
        wid = lax.axis_index("s") * NC + lax.axis_index("c")
        base = wid * BPW
        pltpu.sync_copy(idx_hbm.at[pl.ds(base, BPW)], idx_v)

        @pl.when(wid == 0)
        def _():
            pltpu.sync_copy(idxp_hbm, idxp_v)
            pltpu.async_copy(table_hbm.at[idxp_v], pbuf, gsem).wait()
            pltpu.sync_copy(pbuf, outp_hbm)

        def fire(k, b):
            pltpu.async_copy(
                table_hbm.at[idx_v.at[pl.ds(k * GSZ, GSZ)]], bufs.at[b], gsem)

        def drain_write(k, b):
            pltpu.make_async_copy(
                table_hbm.at[idx_v.at[pl.ds(k * GSZ, GSZ)]], bufs.at[b], gsem).wait()
            pltpu.sync_copy(bufs.at[b], out_hbm.at[pl.ds(base + k * GSZ, GSZ)])

        for b in range(NB - 1):
            fire(b, b)

        def body(kq, carry):
            for b in range(NB):
                k = NB * kq + b

                @pl.when(k + NB - 1 < K)
                def _():
                    fire(k + NB - 1, (k + NB - 1) % NB)

                drain_write(k, b)
            return carry

        lax.fori_loop(0, K // NB, body, 0)
        for k in range(NB * (K // NB), K):
            drain_write(k, k % NB)

    return gather_kernel(emb_table, idx, idx_p)


def _tc_body(emb_ref, seg_ref, prow_ref, pe_ref, w_ref, b_ref, out_ref):
    p0 = jnp.sum(prow_ref[0:40, :], axis=0, keepdims=True)
    p1 = jnp.sum(prow_ref[40:80, :], axis=0, keepdims=True)
    seg = seg_ref[...]
    m0 = (seg == SPE1_IDX).astype(jnp.float32)
    m1 = (seg == SPE2_IDX).astype(jnp.float32)
    emb = emb_ref[...] * np.float32(8.0) + m0 * p0 + m1 * p1 + pe_ref[...]
    out_ref[...] = (
        jnp.dot(emb, w_ref[...], preferred_element_type=jnp.float32) + b_ref[...]
    )


def _tc_project(gathered, seg_col, prows, proj_w, proj_b2, pe_rep):
    return pl.pallas_call(
        _tc_body,
        grid=(GRID,),
        in_specs=[
            pl.BlockSpec((RBLK, EMB_DIM), lambda i: (i, 0)),
            pl.BlockSpec((RBLK, 1), lambda i: (i, 0)),
            pl.BlockSpec((NPROWS, EMB_DIM), lambda i: (0, 0)),
            pl.BlockSpec((RBLK, EMB_DIM), lambda i: (0, 0)),
            pl.BlockSpec((EMB_DIM, 512), lambda i: (0, 0)),
            pl.BlockSpec((1, 512), lambda i: (0, 0)),
        ],
        out_specs=pl.BlockSpec((RBLK, 512), lambda i: (i, 0)),
        out_shape=jax.ShapeDtypeStruct((TOK, 512), jnp.float32),
    )(gathered, seg_col, prows, pe_rep, proj_w, proj_b2)


def kernel(context, segs, personas_no_tag, tags, emb_table, proj_w, proj_b):
    idx_main = context.reshape(-1).astype(jnp.int32)
    idx_p = jnp.concatenate([
        personas_no_tag[0], tags[0],
        personas_no_tag[1], tags[1],
    ]).astype(jnp.int32)

    gathered, prows = _sc_gather(emb_table, idx_main, idx_p)

    seg_col = segs.reshape(TOK, 1).astype(jnp.int32)
    out = _tc_project(gathered, seg_col, prows, proj_w, proj_b.reshape(1, 512),
                      jnp.asarray(_PE_REP))
    return out.reshape(BATCH, SEQ, 512)


# speedup vs baseline: 1.5351x; 1.0309x over previous
import functools

import numpy as np
import jax
import jax.numpy as jnp
from jax import lax
from jax.experimental import pallas as pl
from jax.experimental.pallas import tpu as pltpu
from jax.experimental.pallas import tpu_sc as plsc

EMB_DIM = 64
SPE1_IDX = 2
SPE2_IDX = 3
SEQ = 200
BATCH = 1024
TOK = BATCH * SEQ
NPROWS = 80
VOCAB2 = 500000

NC, NS = 2, 16
NW = NC * NS
GSZ = 128
K = 50
BPW = GSZ * K
NB = 4

RBLK = 3200
GRID = TOK // RBLK


def _positional_encoding(L, d):
    position = np.arange(L, dtype=np.float32)[:, None]
    div_term = np.exp(np.arange(0, d, 2, dtype=np.float32) * (-np.log(10000.0) / d))
    pe = np.zeros((L, d), dtype=np.float32)
    pe[:, 0::2] = np.sin(position * div_term)
    pe[:, 1::2] = np.cos(position * div_term)
    return pe


_PE_REP = np.tile(_positional_encoding(SEQ, EMB_DIM), (RBLK // SEQ, 1))


def _sc_gather(table2, idx2, idxp2):
    mesh = plsc.VectorSubcoreMesh(core_axis_name="c", subcore_axis_name="s")

    @functools.partial(
        pl.kernel,
        mesh=mesh,
        out_type=(
            jax.ShapeDtypeStruct((TOK, 2 * EMB_DIM), jnp.float32),
            jax.ShapeDtypeStruct((NPROWS, 2 * EMB_DIM), jnp.float32),
        ),
        scratch_types=[
            pltpu.VMEM((BPW,), jnp.int32),
            pltpu.VMEM((NB, GSZ, 2 * EMB_DIM), jnp.float32),
            pltpu.VMEM((NPROWS,), jnp.int32),
            pltpu.VMEM((NPROWS, 2 * EMB_DIM), jnp.float32),
            pltpu.SemaphoreType.DMA,
        ],
        compiler_params=pltpu.CompilerParams(use_tc_tiling_on_sc=True),
    )
    def gather_kernel(table_hbm, idx_hbm, idxp_hbm, out_hbm, outp_hbm,
                      idx_v, bufs, idxp_v, pbuf, gsem):
        wid = lax.axis_index("s") * NC + lax.axis_index("c")
        base = wid * BPW
        pltpu.sync_copy(idx_hbm.at[pl.ds(base, BPW)], idx_v)

        @pl.when(wid == 0)
        def _():
            pltpu.sync_copy(idxp_hbm, idxp_v)
            pltpu.async_copy(table_hbm.at[idxp_v], pbuf, gsem).wait()
            pltpu.sync_copy(pbuf, outp_hbm)

        def fire(k, b):
            pltpu.async_copy(
                table_hbm.at[idx_v.at[pl.ds(k * GSZ, GSZ)]], bufs.at[b], gsem)

        def drain_write(k, b):
            pltpu.make_async_copy(
                table_hbm.at[idx_v.at[pl.ds(k * GSZ, GSZ)]], bufs.at[b], gsem).wait()
            pltpu.sync_copy(bufs.at[b], out_hbm.at[pl.ds(base + k * GSZ, GSZ)])

        for b in range(NB - 1):
            fire(b, b)

        def body(kq, carry):
            for b in range(NB):
                k = NB * kq + b

                @pl.when(k + NB - 1 < K)
                def _():
                    fire(k + NB - 1, (k + NB - 1) % NB)

                drain_write(k, b)
            return carry

        lax.fori_loop(0, K // NB, body, 0)
        for k in range(NB * (K // NB), K):
            drain_write(k, k % NB)

    return gather_kernel(table2, idx2, idxp2)


def _tc_body(emb_ref, code_ref, prow_ref, parp_ref, pe_ref, w_ref, b_ref, out_ref):
    parp = (parp_ref[...] & 1).astype(jnp.float32)
    pr = prow_ref[...]
    pr = pr[:, 0:EMB_DIM] * (1.0 - parp) + pr[:, EMB_DIM:] * parp
    p0 = jnp.sum(pr[0:40, :], axis=0, keepdims=True)
    p1 = jnp.sum(pr[40:80, :], axis=0, keepdims=True)

    code = code_ref[...]
    par = (code & 1).astype(jnp.float32)
    seg = code >> 1
    m0 = (seg == SPE1_IDX).astype(jnp.float32)
    m1 = (seg == SPE2_IDX).astype(jnp.float32)

    pairs = emb_ref[...]
    emb = pairs[:, 0:EMB_DIM] * (1.0 - par) + pairs[:, EMB_DIM:] * par
    emb = emb * np.float32(8.0) + m0 * p0 + m1 * p1 + pe_ref[...]
    out_ref[...] = (
        jnp.dot(emb, w_ref[...], preferred_element_type=jnp.float32) + b_ref[...]
    )


def _tc_project(pairs, code_col, prows, parp, proj_w, proj_b2, pe_rep):
    return pl.pallas_call(
        _tc_body,
        grid=(GRID,),
        in_specs=[
            pl.BlockSpec((RBLK, 2 * EMB_DIM), lambda i: (i, 0)),
            pl.BlockSpec((RBLK, 1), lambda i: (i, 0)),
            pl.BlockSpec((NPROWS, 2 * EMB_DIM), lambda i: (0, 0)),
            pl.BlockSpec((NPROWS, 1), lambda i: (0, 0)),
            pl.BlockSpec((RBLK, EMB_DIM), lambda i: (0, 0)),
            pl.BlockSpec((EMB_DIM, 512), lambda i: (0, 0)),
            pl.BlockSpec((1, 512), lambda i: (0, 0)),
        ],
        out_specs=pl.BlockSpec((RBLK, 512), lambda i: (i, 0)),
        out_shape=jax.ShapeDtypeStruct((TOK, 512), jnp.float32),
    )(pairs, code_col, prows, parp, pe_rep, proj_w, proj_b2)


def kernel(context, segs, personas_no_tag, tags, emb_table, proj_w, proj_b):
    table2 = emb_table.reshape(VOCAB2, 2 * EMB_DIM)
    ctx = context.astype(jnp.int32)
    idx2 = (ctx >> 1).reshape(-1)
    code_col = (segs.astype(jnp.int32) * 2 + (ctx & 1)).reshape(TOK, 1)

    idx_p = jnp.concatenate([
        personas_no_tag[0], tags[0],
        personas_no_tag[1], tags[1],
    ]).astype(jnp.int32)
    idxp2 = idx_p >> 1
    parp = (idx_p & 1).reshape(NPROWS, 1)

    pairs, prows = _sc_gather(table2, idx2, idxp2)

    out = _tc_project(pairs, code_col, prows, parp, proj_w,
                      proj_b.reshape(1, 512), jnp.asarray(_PE_REP))
    return out.reshape(BATCH, SEQ, 512)
